# Initial kernel scaffold; baseline (speedup 1.0000x reference)
import jax
import jax.numpy as jnp
from jax.experimental import pallas as pl


def kernel(x, params, edge_index, batch):
    raise NotImplementedError("write your pallas kernel here")


# speedup vs baseline: 4.7827x; 4.7827x over previous
import functools

import jax
import jax.numpy as jnp
from jax import lax
from jax.experimental import pallas as pl
from jax.experimental.pallas import tpu as pltpu
from jax.experimental.pallas import tpu_sc as plsc

N = 10000
E = 320000
D_IN = 128
D_H = 32
G = 128

NC = 2
NS = 16
NW = NC * NS
EPW = E // NW
CH = 128
NCH = 80
EPW_PAD = NCH * CH
RPT = 632
ZR = RPT // 4
N_ACC = RPT * NS


@functools.cache
def _build_segsum_sc(d: int, k: int):
    mesh = plsc.VectorSubcoreMesh(core_axis_name="c", subcore_axis_name="s",
                                  num_cores=NC, num_subcores=NS)

    def body(srcp, dstp, h, out, sidx, didx, rows, zbuf, acc, gsems):
        cid = lax.axis_index("c")
        sid = lax.axis_index("s")
        worker = sid * NC + cid

        pltpu.sync_copy(srcp.at[worker], sidx)
        pltpu.sync_copy(dstp.at[worker], didx)

        def _zrow(r, carry):
            for c in range(d // 16):
                zbuf[r, pl.ds(16 * c, 16)] = jnp.zeros((16,), jnp.float32)
            return carry

        lax.fori_loop(0, ZR, _zrow, 0)
        for q in range(4):
            pltpu.sync_copy(zbuf, acc.at[pl.ds(sid * RPT + q * ZR, ZR)])
        plsc.subcore_barrier()

        def _group(g, carry):
            base = g * k
            gets = [pltpu.async_copy(h.at[sidx.at[base + b]], rows.at[b],
                                     gsems[b]) for b in range(k)]
            for b in range(k):
                gets[b].wait()
                pltpu.sync_copy(rows.at[b], acc.at[didx.at[base + b]],
                                add=True)
            return carry

        lax.fori_loop(0, NCH // k, _group, 0)
        plsc.subcore_barrier()

        pltpu.sync_copy(acc.at[pl.ds(sid * RPT, RPT)],
                        out.at[cid, pl.ds(sid * RPT, RPT)])

    return functools.partial(
        pl.kernel,
        mesh=mesh,
        out_type=jax.ShapeDtypeStruct((NC, N_ACC, d), jnp.float32),
        scratch_types=[
            pltpu.VMEM((NCH, CH), jnp.int32),
            pltpu.VMEM((NCH, CH), jnp.int32),
            pltpu.VMEM((k, CH, d), jnp.float32),
            pltpu.VMEM((ZR, d), jnp.float32),
            pltpu.VMEM_SHARED((N_ACC, d), jnp.float32),
            [pltpu.SemaphoreType.DMA] * k,
        ],
        compiler_params=pltpu.CompilerParams(use_tc_tiling_on_sc=False),
    )(body)


def _mlp_bn(z, w1_ref, b1_ref, w2_ref, b2_ref, g_ref, be_ref):
    z = jnp.maximum(jnp.dot(z, w1_ref[...],
                            preferred_element_type=jnp.float32)
                    + b1_ref[...], 0.0)
    z = jnp.dot(z, w2_ref[...], preferred_element_type=jnp.float32) \
        + b2_ref[...]
    h = jnp.maximum(z, 0.0)
    mean = jnp.mean(h, axis=0, keepdims=True)
    c = h - mean
    var = jnp.mean(c * c, axis=0, keepdims=True)
    return c / jnp.sqrt(var + 1e-5) * g_ref[...] + be_ref[...]


def _tc_first_body(h_ref, pa_ref, pb_ref, w1_ref, b1_ref, w2_ref, b2_ref,
                   g_ref, be_ref, out_ref):
    agg = jnp.concatenate([pa_ref[0, :N, :] + pa_ref[1, :N, :],
                           pb_ref[0, :N, :] + pb_ref[1, :N, :]], axis=1)
    out_ref[...] = _mlp_bn(h_ref[...] + agg, w1_ref, b1_ref, w2_ref, b2_ref,
                           g_ref, be_ref)


def _tc_mid_body(h_ref, p_ref, w1_ref, b1_ref, w2_ref, b2_ref, g_ref, be_ref,
                 out_ref):
    z = h_ref[...] + (p_ref[0, :N, :] + p_ref[1, :N, :])
    out_ref[...] = _mlp_bn(z, w1_ref, b1_ref, w2_ref, b2_ref, g_ref, be_ref)


def _tc_fin_body(h_ref, p_ref, w1_ref, b1_ref, w2_ref, b2_ref, g_ref, be_ref,
                 batch_ref, wf_ref, bf_ref, out_ref):
    z = h_ref[...] + (p_ref[0, :N, :] + p_ref[1, :N, :])
    hn = _mlp_bn(z, w1_ref, b1_ref, w2_ref, b2_ref, g_ref, be_ref)
    onehot = (batch_ref[...] ==
              lax.broadcasted_iota(jnp.int32, (G, N), 0)).astype(jnp.float32)
    pooled = jnp.dot(onehot, hn, preferred_element_type=jnp.float32)
    out_ref[...] = jnp.maximum(
        jnp.dot(pooled, wf_ref[...], preferred_element_type=jnp.float32)
        + bf_ref[...], 0.0)


_tc_first = pl.pallas_call(
    _tc_first_body, out_shape=jax.ShapeDtypeStruct((N, D_H), jnp.float32))
_tc_mid = pl.pallas_call(
    _tc_mid_body, out_shape=jax.ShapeDtypeStruct((N, D_H), jnp.float32))
_tc_fin = pl.pallas_call(
    _tc_fin_body, out_shape=jax.ShapeDtypeStruct((G, D_IN), jnp.float32))


def kernel(x, params, edge_index, batch):
    order = jnp.argsort(edge_index[1], stable=True)
    src = edge_index[0][order]
    dst = edge_index[1][order]
    pad = jnp.zeros((NW, EPW_PAD - EPW), jnp.int32)
    srcp = jnp.concatenate([src.reshape(NW, EPW), pad], axis=1)
    srcp = srcp.reshape(NW, NCH, CH)
    dstp = jnp.concatenate([dst.reshape(NW, EPW), pad + N], axis=1)
    dstp = dstp.reshape(NW, NCH, CH)
    batch_row = batch.reshape(1, N)

    h = x
    for i in range(1, 6):
        cp = params['conv%d' % i]
        bp = params['bn%d' % i]
        wb = (cp['w1'], cp['b1'].reshape(1, D_H), cp['w2'],
              cp['b2'].reshape(1, D_H), bp['gamma'].reshape(1, D_H),
              bp['beta'].reshape(1, D_H))
        if i == 1:
            seg = _build_segsum_sc(D_IN // 2, 4)
            pa = seg(srcp, dstp, h[:, :D_IN // 2])
            pb = seg(srcp, dstp, h[:, D_IN // 2:])
            h = _tc_first(h, pa, pb, *wb)
        else:
            p = _build_segsum_sc(D_H, 8)(srcp, dstp, h)
            if i < 5:
                h = _tc_mid(h, p, *wb)
            else:
                out = _tc_fin(h, p, *wb, batch_row, params['fc1']['w'],
                              params['fc1']['b'].reshape(1, D_IN))
    return out
